# baseline (speedup 1.0000x reference)
import functools

import jax
import jax.numpy as jnp
from jax import lax
from jax.experimental import pallas as pl
from jax.experimental.pallas import tpu as pltpu
from jax.experimental.pallas import tpu_sc as plsc

_B = 4096 * 200
_D = 64
_NC, _NS = 2, 16
_NW = _NC * _NS
_BPW = _B // _NW
_C = 512
_NCHUNK = _BPW // _C

_mesh = plsc.VectorSubcoreMesh(core_axis_name="c", subcore_axis_name="s")


@functools.partial(
    pl.kernel,
    out_type=jax.ShapeDtypeStruct((_B, _D), jnp.float32),
    mesh=_mesh,
    scratch_types=[
        pltpu.VMEM((_C,), jnp.int32),
        pltpu.VMEM((_C, _D), jnp.float32),
        pltpu.SemaphoreType.DMA,
    ],
    compiler_params=pltpu.CompilerParams(use_tc_tiling_on_sc=False),
)
def _embed(idx_hbm, table_hbm, out_hbm, idx_v, rows_v, sem):
    wid = lax.axis_index("s") * _NC + lax.axis_index("c")
    wbase = wid * _BPW

    def body(g, carry):
        base = wbase + g * _C
        pltpu.sync_copy(idx_hbm.at[pl.ds(base, _C)], idx_v)
        pltpu.async_copy(table_hbm.at[idx_v], rows_v, sem).wait()
        pltpu.sync_copy(rows_v, out_hbm.at[pl.ds(base, _C)])
        return carry

    lax.fori_loop(0, _NCHUNK, body, 0)


def kernel(inputs, table):
    idx = inputs.reshape(-1).astype(jnp.int32)
    out = _embed(idx, table)
    return out.reshape(inputs.shape + (table.shape[1],))


# speedup vs baseline: 1.0376x; 1.0376x over previous
import functools

import jax
import jax.numpy as jnp
from jax import lax
from jax.experimental import pallas as pl
from jax.experimental.pallas import tpu as pltpu
from jax.experimental.pallas import tpu_sc as plsc

_B = 4096 * 200
_D = 64
_NC, _NS = 2, 16
_NW = _NC * _NS
_BPW = _B // _NW
_C = 800
_NCHUNK = _BPW // _C
_K = _NCHUNK // 2

_mesh = plsc.VectorSubcoreMesh(core_axis_name="c", subcore_axis_name="s")


@functools.partial(
    pl.kernel,
    out_type=jax.ShapeDtypeStruct((_B, _D), jnp.float32),
    mesh=_mesh,
    scratch_types=[
        pltpu.VMEM((_C,), jnp.int32),
        pltpu.VMEM((_C,), jnp.int32),
        pltpu.VMEM((_C, _D), jnp.float32),
        pltpu.VMEM((_C, _D), jnp.float32),
        pltpu.SemaphoreType.DMA,
        pltpu.SemaphoreType.DMA,
        pltpu.SemaphoreType.DMA,
        pltpu.SemaphoreType.DMA,
        pltpu.SemaphoreType.DMA,
        pltpu.SemaphoreType.DMA,
    ],
    compiler_params=pltpu.CompilerParams(use_tc_tiling_on_sc=False),
)
def _embed(idx_hbm, table_hbm, out_hbm, idx0, idx1, rows0, rows1,
           isem0, isem1, gsem0, gsem1, ssem0, ssem1):
    wid = lax.axis_index("s") * _NC + lax.axis_index("c")
    wbase = wid * _BPW

    def idx_start(g, buf, sem):
        return pltpu.async_copy(idx_hbm.at[pl.ds(wbase + g * _C, _C)], buf, sem)

    def gather_start(buf_idx, buf_rows, sem):
        return pltpu.async_copy(table_hbm.at[buf_idx], buf_rows, sem)

    def store_start(g, buf_rows, sem):
        return pltpu.async_copy(buf_rows, out_hbm.at[pl.ds(wbase + g * _C, _C)], sem)

    c_i0 = idx_start(0, idx0, isem0)
    idx_start(1, idx1, isem1)
    c_i0.wait()
    gather_start(idx0, rows0, gsem0)

    def body(k, carry):
        g0 = 2 * k
        pltpu.make_async_copy(table_hbm.at[idx0], rows0, gsem0).wait()
        store_start(g0, rows0, ssem0)

        @pl.when(k > 0)
        def _():
            pltpu.make_async_copy(rows1, out_hbm.at[pl.ds(wbase, _C)], ssem1).wait()

        @pl.when(k < _K - 1)
        def _():
            idx_start(g0 + 2, idx0, isem0)

        pltpu.make_async_copy(idx_hbm.at[pl.ds(wbase, _C)], idx1, isem1).wait()
        gather_start(idx1, rows1, gsem1)
        pltpu.make_async_copy(table_hbm.at[idx1], rows1, gsem1).wait()
        store_start(g0 + 1, rows1, ssem1)

        @pl.when(k < _K - 1)
        def _():
            idx_start(g0 + 3, idx1, isem1)
            pltpu.make_async_copy(rows0, out_hbm.at[pl.ds(wbase, _C)], ssem0).wait()
            pltpu.make_async_copy(idx_hbm.at[pl.ds(wbase, _C)], idx0, isem0).wait()
            gather_start(idx0, rows0, gsem0)

        return carry

    lax.fori_loop(0, _K, body, 0)

    pltpu.make_async_copy(rows0, out_hbm.at[pl.ds(wbase, _C)], ssem0).wait()
    pltpu.make_async_copy(rows1, out_hbm.at[pl.ds(wbase, _C)], ssem1).wait()


def kernel(inputs, table):
    idx = inputs.reshape(-1).astype(jnp.int32)
    out = _embed(idx, table)
    return out.reshape(inputs.shape + (table.shape[1],))
